# baseline (speedup 1.0000x reference)
import functools

import numpy as np
import jax
import jax.numpy as jnp
from jax import lax
from jax.experimental import pallas as pl
from jax.experimental.pallas import tpu as pltpu
from jax.experimental.pallas import tpu_sc as plsc

INPUT_DIM = 3
NUM_LEVELS = 16
LEVEL_DIM = 2
BASE_RES = 16
LOG2_T = 19
T = 2 ** LOG2_T
DESIRED_RES = 2048
N_POINTS = 524288

_b = np.exp(np.log(DESIRED_RES / BASE_RES) / (NUM_LEVELS - 1))
RES_LIST = []
OFFSETS = [0]
for _l in range(NUM_LEVELS):
    _res = int(np.ceil(BASE_RES * (_b ** _l)))
    RES_LIST.append(_res)
    OFFSETS.append(OFFSETS[-1] + min(T, (_res + 1) ** INPUT_DIM))
TOTAL_ROWS = OFFSETS[-1]
NUM_DENSE = sum(1 for r in RES_LIST if (r + 1) ** 3 <= T)
assert all(OFFSETS[l + 1] - OFFSETS[l] == T for l in range(NUM_DENSE, NUM_LEVELS))

P1 = int(np.uint32(2654435761).astype(np.int64)) - (1 << 32)
P2 = 805459861
MASK = T - 1

NC, NS, LANES = 2, 16, 16
NW = NC * NS
PTS_PER_W = N_POINTS // NW
C = 64
PAD_D = 8
CHUNKS = PTS_PER_W // C
R = NUM_LEVELS * 8

def _build(n_points, interpret=False):
  pts_per_w = n_points // NW
  chunks = pts_per_w // C
  out_len = n_points * NUM_LEVELS * LEVEL_DIM

  @functools.partial(
      pl.kernel,
      out_type=jax.ShapeDtypeStruct((out_len,), jnp.float32),
      mesh=plsc.VectorSubcoreMesh(core_axis_name="c", subcore_axis_name="s",
                                  num_cores=NC, num_subcores=NS),
      scratch_types=[
          pltpu.VMEM((C * INPUT_DIM,), jnp.float32),
          pltpu.VMEM((2 * R * C,), jnp.int32),
          pltpu.VMEM((R * C,), jnp.float32),
          pltpu.VMEM((2 * R * C,), jnp.float32),
          pltpu.VMEM((C * NUM_LEVELS * LEVEL_DIM,), jnp.float32),
          pltpu.VMEM((LANES,), jnp.float32),
          pltpu.VMEM((LANES,), jnp.int32),
          pltpu.SemaphoreType.DMA,
      ],
      compiler_params=pltpu.CompilerParams(needs_layout_passes=False,
                                           use_tc_tiling_on_sc=False),
      interpret=interpret,
  )
  def _sc_encode(x_hbm, emb_hbm, res_hbm, off_hbm, out_hbm,
                 x_buf, idx_buf, w_buf, rows_buf, out_buf, res_buf, off_buf,
                 gsem):
    cid = lax.axis_index("c")
    sid = lax.axis_index("s")
    wid = sid * NC + cid
    pltpu.sync_copy(res_hbm, res_buf)
    pltpu.sync_copy(off_hbm, off_buf)
    iota = lax.iota(jnp.int32, LANES)
    zerov = jnp.zeros((LANES,), jnp.int32)
    onev = zerov + 1

    def chunk_body(ci, _):
        base = wid * pts_per_w + ci * C
        pltpu.sync_copy(x_hbm.at[pl.ds(base * INPUT_DIM, C * INPUT_DIM)],
                        x_buf)

        def gen_group(g, _):
            pvec = g * LANES + iota
            pv3 = pvec * INPUT_DIM
            xs = [plsc.load_gather(x_buf, [pv3 + d]) for d in range(3)]

            for l in range(NUM_DENSE):
                lres = RES_LIST[l]
                vdim = lres + 1
                resv = jnp.full((LANES,), np.float32(lres), jnp.float32)
                pos = [x * resv for x in xs]
                ip = [p.astype(jnp.int32) for p in pos]
                frac = [p - i.astype(jnp.float32) for p, i in zip(pos, ip)]
                p0 = [jnp.minimum(i, lres - 1) for i in ip]
                om = [1.0 - f for f in frac]
                a0 = p0[0] * (2 * vdim * vdim) + (2 * OFFSETS[l])
                a1 = a0 + 2 * vdim * vdim
                b0 = p0[1] * (2 * vdim)
                b1 = b0 + 2 * vdim
                c0 = p0[2] * 2
                c1 = c0 + 2
                pvec2 = pvec * 2
                for j in range(8):
                    bx, by, bz = j & 1, (j >> 1) & 1, (j >> 2) & 1
                    idx2 = ((a1 if bx else a0) + (b1 if by else b0)
                            + (c1 if bz else c0))
                    w = (frac[0] if bx else om[0]) * (frac[1] if by else om[1])
                    w = w * (frac[2] if bz else om[2])
                    plsc.store_scatter(w_buf, [pvec + ((l * 8 + j) * C)], w)
                    rsp2 = pvec2 + ((l * 8 + j) * (2 * C))
                    plsc.store_scatter(idx_buf, [rsp2], idx2)
                    plsc.store_scatter(idx_buf, [rsp2 + 1], idx2 + 1)

            def hash_level(l, _):
                lv = zerov + l
                resv = plsc.load_gather(res_buf, [lv])
                offv = plsc.load_gather(off_buf, [lv])
                pos = [x * resv for x in xs]
                ip = [p.astype(jnp.int32) for p in pos]
                frac = [p - i.astype(jnp.float32) for p, i in zip(pos, ip)]
                rm1 = resv.astype(jnp.int32) - 1
                p0 = [jnp.minimum(i, m) for i, m in zip(ip, [rm1] * 3)]
                om = [1.0 - f for f in frac]
                a0 = p0[0]
                a1 = a0 + 1
                b0 = p0[1] * P1
                b1 = b0 + P1
                c0 = p0[2] * P2
                c1 = c0 + P2
                pl8 = pvec + l * (8 * C)
                pl8b = pvec * 2 + l * (16 * C)
                off2 = offv * 2
                for j in range(8):
                    bx, by, bz = j & 1, (j >> 1) & 1, (j >> 2) & 1
                    h = (a1 if bx else a0) ^ (b1 if by else b0)
                    h = h ^ (c1 if bz else c0)
                    idx2 = (h & MASK) * 2 + off2
                    w = (frac[0] if bx else om[0]) * (frac[1] if by else om[1])
                    w = w * (frac[2] if bz else om[2])
                    plsc.store_scatter(w_buf, [pl8 + (j * C)], w)
                    rsp2 = pl8b + (j * (2 * C))
                    plsc.store_scatter(idx_buf, [rsp2], idx2)
                    plsc.store_scatter(idx_buf, [rsp2 + 1], idx2 + 1)
                return 0

            lax.fori_loop(NUM_DENSE, NUM_LEVELS, hash_level, 0)
            return 0

        lax.fori_loop(0, C // LANES, gen_group, 0)

        half = R * C
        pltpu.async_copy(emb_hbm.at[idx_buf.at[pl.ds(0, half)]],
                         rows_buf.at[pl.ds(0, half)], gsem)
        pltpu.async_copy(emb_hbm.at[idx_buf.at[pl.ds(half, half)]],
                         rows_buf.at[pl.ds(half, half)], gsem)
        pltpu.make_async_copy(emb_hbm.at[idx_buf.at[pl.ds(0, half)]],
                              rows_buf.at[pl.ds(0, half)], gsem).wait()
        pltpu.make_async_copy(emb_hbm.at[idx_buf.at[pl.ds(half, half)]],
                              rows_buf.at[pl.ds(half, half)], gsem).wait()

        def mac_group(g, _):
            pvec = g * LANES + iota
            pw = pvec * (NUM_LEVELS * LEVEL_DIM)

            def mac_level(l, _):
                rb = pvec + l * (8 * C)
                rb2 = pvec * 2 + l * (16 * C)
                acc0 = jnp.zeros((LANES,), jnp.float32)
                acc1 = jnp.zeros((LANES,), jnp.float32)
                for j in range(8):
                    rv = rb + (j * C)
                    rv2 = rb2 + (j * (2 * C))
                    wv = plsc.load_gather(w_buf, [rv])
                    g0 = plsc.load_gather(rows_buf, [rv2])
                    g1 = plsc.load_gather(rows_buf, [rv2 + 1])
                    acc0 = acc0 + g0 * wv
                    acc1 = acc1 + g1 * wv
                s0 = pw + 2 * l
                plsc.store_scatter(out_buf, [s0], acc0)
                plsc.store_scatter(out_buf, [s0 + 1], acc1)
                return 0

            lax.fori_loop(0, NUM_LEVELS, mac_level, 0)
            return 0

        lax.fori_loop(0, C // LANES, mac_group, 0)
        pltpu.sync_copy(out_buf,
                        out_hbm.at[pl.ds(base * (NUM_LEVELS * LEVEL_DIM),
                                         C * NUM_LEVELS * LEVEL_DIM)])
        return 0

    lax.fori_loop(0, chunks, chunk_body, 0)

  return _sc_encode


_sc_encode_full = _build(N_POINTS)


def kernel(x, embeddings):
    res_arr = jnp.asarray(np.array(RES_LIST, np.float32))
    off_arr = jnp.asarray(np.array(OFFSETS[:NUM_LEVELS], np.int32))
    out = _sc_encode_full(x.reshape(-1), embeddings.reshape(-1),
                          res_arr, off_arr)
    return out.reshape(N_POINTS, NUM_LEVELS * LEVEL_DIM)


# speedup vs baseline: 2.1881x; 2.1881x over previous
import functools

import numpy as np
import jax
import jax.numpy as jnp
from jax import lax
from jax.experimental import pallas as pl
from jax.experimental.pallas import tpu as pltpu
from jax.experimental.pallas import tpu_sc as plsc

INPUT_DIM = 3
NUM_LEVELS = 16
LEVEL_DIM = 2
BASE_RES = 16
LOG2_T = 19
T = 2 ** LOG2_T
DESIRED_RES = 2048
N_POINTS = 524288

_b = np.exp(np.log(DESIRED_RES / BASE_RES) / (NUM_LEVELS - 1))
RES_LIST = []
OFFSETS = [0]
for _l in range(NUM_LEVELS):
    _res = int(np.ceil(BASE_RES * (_b ** _l)))
    RES_LIST.append(_res)
    OFFSETS.append(OFFSETS[-1] + min(T, (_res + 1) ** INPUT_DIM))
TOTAL_ROWS = OFFSETS[-1]
NUM_DENSE = sum(1 for r in RES_LIST if (r + 1) ** 3 <= T)
assert all(OFFSETS[l + 1] - OFFSETS[l] == T for l in range(NUM_DENSE, NUM_LEVELS))

P1 = int(np.uint32(2654435761).astype(np.int64)) - (1 << 32)
P2 = 805459861
MASK = T - 1

NC, NS, LANES = 2, 16, 16
NW = NC * NS
C = 128
R = NUM_LEVELS * 8


def _build(n_points, interpret=False):
  pts_per_w = n_points // NW
  chunks = pts_per_w // C
  out_len = n_points * NUM_LEVELS * LEVEL_DIM

  @functools.partial(
      pl.kernel,
      out_type=jax.ShapeDtypeStruct((out_len,), jnp.float32),
      mesh=plsc.VectorSubcoreMesh(core_axis_name="c", subcore_axis_name="s",
                                  num_cores=NC, num_subcores=NS),
      scratch_types=[
          pltpu.VMEM((C,), jnp.float32),
          pltpu.VMEM((C,), jnp.float32),
          pltpu.VMEM((C,), jnp.float32),
          pltpu.VMEM((R * C,), jnp.int32),
          pltpu.VMEM((R * C,), jnp.float32),
          pltpu.VMEM((R * C,), jnp.float32),
          pltpu.VMEM((R * C,), jnp.float32),
          pltpu.VMEM((C * NUM_LEVELS * LEVEL_DIM,), jnp.float32),
          pltpu.VMEM((LANES,), jnp.float32),
          pltpu.VMEM((LANES,), jnp.int32),
          pltpu.SemaphoreType.DMA,
      ],
      compiler_params=pltpu.CompilerParams(needs_layout_passes=False,
                                           use_tc_tiling_on_sc=False),
      interpret=interpret,
  )
  def _sc_encode(x0_hbm, x1_hbm, x2_hbm, f0_hbm, f1_hbm, res_hbm, off_hbm,
                 out_hbm,
                 x0_buf, x1_buf, x2_buf, idx_buf, w_buf, rows0, rows1,
                 out_buf, res_buf, off_buf, gsem):
    cid = lax.axis_index("c")
    sid = lax.axis_index("s")
    wid = sid * NC + cid
    pltpu.sync_copy(res_hbm, res_buf)
    pltpu.sync_copy(off_hbm, off_buf)
    iota = lax.iota(jnp.int32, LANES)
    zerov = jnp.zeros((LANES,), jnp.int32)

    def chunk_body(ci, _):
        base = wid * pts_per_w + ci * C
        pltpu.sync_copy(x0_hbm.at[pl.ds(base, C)], x0_buf)
        pltpu.sync_copy(x1_hbm.at[pl.ds(base, C)], x1_buf)
        pltpu.sync_copy(x2_hbm.at[pl.ds(base, C)], x2_buf)

        def gen_group(g, _):
            pvec = g * LANES + iota
            gsl = pl.ds(g * LANES, LANES)
            xs = [x0_buf[gsl], x1_buf[gsl], x2_buf[gsl]]

            for l in range(NUM_DENSE):
                lres = RES_LIST[l]
                vdim = lres + 1
                resv = jnp.full((LANES,), np.float32(lres), jnp.float32)
                pos = [x * resv for x in xs]
                ip = [p.astype(jnp.int32) for p in pos]
                frac = [p - i.astype(jnp.float32) for p, i in zip(pos, ip)]
                p0 = [jnp.minimum(i, lres - 1) for i in ip]
                om = [1.0 - f for f in frac]
                a0 = p0[0] * (vdim * vdim) + OFFSETS[l]
                a1 = a0 + vdim * vdim
                b0 = p0[1] * vdim
                b1 = b0 + vdim
                c0 = p0[2]
                c1 = c0 + 1
                for j in range(8):
                    bx, by, bz = j & 1, (j >> 1) & 1, (j >> 2) & 1
                    idx = ((a1 if bx else a0) + (b1 if by else b0)
                           + (c1 if bz else c0))
                    w = (frac[0] if bx else om[0]) * (frac[1] if by else om[1])
                    w = w * (frac[2] if bz else om[2])
                    rsp = pvec + ((l * 8 + j) * C)
                    plsc.store_scatter(idx_buf, [rsp], idx)
                    plsc.store_scatter(w_buf, [rsp], w)

            def hash_level(l, _):
                lv = zerov + l
                resv = plsc.load_gather(res_buf, [lv])
                offv = plsc.load_gather(off_buf, [lv])
                pos = [x * resv for x in xs]
                ip = [p.astype(jnp.int32) for p in pos]
                frac = [p - i.astype(jnp.float32) for p, i in zip(pos, ip)]
                rm1 = resv.astype(jnp.int32) - 1
                p0 = [jnp.minimum(i, m) for i, m in zip(ip, [rm1] * 3)]
                om = [1.0 - f for f in frac]
                a0 = p0[0]
                a1 = a0 + 1
                b0 = p0[1] * P1
                b1 = b0 + P1
                c0 = p0[2] * P2
                c1 = c0 + P2
                pl8 = pvec + l * (8 * C)
                for j in range(8):
                    bx, by, bz = j & 1, (j >> 1) & 1, (j >> 2) & 1
                    h = (a1 if bx else a0) ^ (b1 if by else b0)
                    h = h ^ (c1 if bz else c0)
                    idx = (h & MASK) + offv
                    w = (frac[0] if bx else om[0]) * (frac[1] if by else om[1])
                    w = w * (frac[2] if bz else om[2])
                    rsp = pl8 + (j * C)
                    plsc.store_scatter(idx_buf, [rsp], idx)
                    plsc.store_scatter(w_buf, [rsp], w)
                return 0

            lax.fori_loop(NUM_DENSE, NUM_LEVELS, hash_level, 0)
            return 0

        lax.fori_loop(0, C // LANES, gen_group, 0)

        pltpu.async_copy(f0_hbm.at[idx_buf], rows0, gsem)
        pltpu.async_copy(f1_hbm.at[idx_buf], rows1, gsem)
        pltpu.make_async_copy(f0_hbm.at[idx_buf], rows0, gsem).wait()
        pltpu.make_async_copy(f1_hbm.at[idx_buf], rows1, gsem).wait()

        def mac_group(g, _):
            pvec = g * LANES + iota
            pw = pvec * (NUM_LEVELS * LEVEL_DIM)

            def mac_level(l, _):
                rb = pvec + l * (8 * C)
                acc0 = jnp.zeros((LANES,), jnp.float32)
                acc1 = jnp.zeros((LANES,), jnp.float32)
                for j in range(8):
                    rv = rb + (j * C)
                    wv = plsc.load_gather(w_buf, [rv])
                    g0 = plsc.load_gather(rows0, [rv])
                    g1 = plsc.load_gather(rows1, [rv])
                    acc0 = acc0 + g0 * wv
                    acc1 = acc1 + g1 * wv
                s0 = pw + 2 * l
                plsc.store_scatter(out_buf, [s0], acc0)
                plsc.store_scatter(out_buf, [s0 + 1], acc1)
                return 0

            lax.fori_loop(0, NUM_LEVELS, mac_level, 0)
            return 0

        lax.fori_loop(0, C // LANES, mac_group, 0)
        pltpu.sync_copy(out_buf,
                        out_hbm.at[pl.ds(base * (NUM_LEVELS * LEVEL_DIM),
                                         C * NUM_LEVELS * LEVEL_DIM)])
        return 0

    lax.fori_loop(0, chunks, chunk_body, 0)

  return _sc_encode


_sc_encode_full = _build(N_POINTS)


def kernel(x, embeddings):
    res_arr = jnp.asarray(np.array(RES_LIST, np.float32))
    off_arr = jnp.asarray(np.array(OFFSETS[:NUM_LEVELS], np.int32))
    out = _sc_encode_full(x[:, 0], x[:, 1], x[:, 2],
                          embeddings[:, 0], embeddings[:, 1],
                          res_arr, off_arr)
    return out.reshape(N_POINTS, NUM_LEVELS * LEVEL_DIM)
